# baseline (speedup 1.0000x reference)
import functools

import jax
import jax.numpy as jnp
from jax import lax
from jax.experimental import pallas as pl
from jax.experimental.pallas import tpu as pltpu
from jax.experimental.pallas import tpu_sc as plsc

DIM = 64
PAD_IDX = 0
B = 4096
L = 200
TOTAL = B * L
NC, NS, LANES = 2, 16, 16
NW = NC * NS
PER_W = TOTAL // NW
CHUNK = 128
CHUNKS = PER_W // CHUNK
NBUF = 4


@functools.partial(
    pl.kernel,
    out_type=jax.ShapeDtypeStruct((TOTAL, DIM), jnp.float32),
    mesh=plsc.VectorSubcoreMesh(core_axis_name="c", subcore_axis_name="s"),
    compiler_params=pltpu.CompilerParams(use_tc_tiling_on_sc=False),
    scratch_types=(
        [pltpu.VMEM((CHUNKS, CHUNK), jnp.int32)]
        + [pltpu.VMEM((CHUNK, DIM), jnp.float32) for _ in range(NBUF)]
        + [pltpu.SemaphoreType.DMA for _ in range(2 * NBUF)]
    ),
)
def _emb_gather(idx_hbm, table_hbm, out_hbm, idx_v, *rest):
    rows = rest[:NBUF]
    gsem = rest[NBUF:2 * NBUF]
    wsem = rest[2 * NBUF:3 * NBUF]

    wid = lax.axis_index("s") * NC + lax.axis_index("c")
    row0 = wid * CHUNKS

    pltpu.sync_copy(idx_hbm.at[pl.ds(row0, CHUNKS)], idx_v)

    def g_copy(j, b):
        return pltpu.make_async_copy(table_hbm.at[idx_v.at[j]], rows[b], gsem[b])

    def w_copy(j, b):
        return pltpu.make_async_copy(
            rows[b], out_hbm.at[pl.ds((row0 + j) * CHUNK, CHUNK)], wsem[b])

    for b in range(NBUF):
        g_copy(b, b).start()

    zero16 = jnp.zeros((LANES,), jnp.float32)
    iota16 = lax.iota(jnp.int32, LANES)

    def _lane_min(r):
        for sh in (8, 4, 2, 1):
            perm = jnp.bitwise_xor(iota16, sh).reshape(LANES, 1)
            g = lax.gather(
                r, perm,
                lax.GatherDimensionNumbers(
                    offset_dims=(), collapsed_slice_dims=(0,),
                    start_index_map=(0,)),
                slice_sizes=(1,),
                mode=lax.GatherScatterMode.PROMISE_IN_BOUNDS)
            r = jnp.minimum(r, g)
        return r[0]

    def step(g, carry):
        for b in range(NBUF):
            j = g * NBUF + b
            g_copy(j, b).wait()
            groups = [idx_v[j, pl.ds(grp * LANES, LANES)]
                      for grp in range(CHUNK // LANES)]
            cmin = groups[0]
            for idxg in groups[1:]:
                cmin = jnp.minimum(cmin, idxg)

            @pl.when(_lane_min(cmin) == PAD_IDX)
            def _fix(j=j, b=b):
                def zero_group(grp, c):
                    idxg = idx_v[j, pl.ds(grp * LANES, LANES)]
                    for lane in range(LANES):
                        @pl.when(idxg[lane] == PAD_IDX)
                        def _(r=grp * LANES + lane):
                            for col in range(0, DIM, LANES):
                                rows[b][r, pl.ds(col, LANES)] = zero16
                    return c

                lax.fori_loop(0, CHUNK // LANES, zero_group, 0)
            w_copy(j, b).start()

            @pl.when(j < CHUNKS - NBUF)
            def _next(j=j, b=b):
                w_copy(j, b).wait()
                g_copy(j + NBUF, b).start()
        return carry

    lax.fori_loop(0, CHUNKS // NBUF, step, 0)
    for b in range(NBUF):
        w_copy(CHUNKS - NBUF + b, b).wait()


def kernel(source, table):
    idx = source.astype(jnp.int32).reshape(NW * CHUNKS, CHUNK)
    out = _emb_gather(idx, table)
    return out.reshape(B, L, DIM)


# speedup vs baseline: 1.6325x; 1.6325x over previous
import functools

import jax
import jax.numpy as jnp
from jax import lax
from jax.experimental import pallas as pl
from jax.experimental.pallas import tpu as pltpu
from jax.experimental.pallas import tpu_sc as plsc

VOCAB = 1000000
DIM = 64
PAD_IDX = 0
B = 4096
L = 200
TOTAL = B * L
NC, NS, LANES = 2, 16, 16
NW = NC * NS
PER_W = TOTAL // NW
IDXW = 128
IDXR = 2
CHUNK = IDXR * IDXW
CHUNKS = PER_W // CHUNK
IROWS = PER_W // IDXW
NBUF = 4


PAIR = 12800
N_TBLK = -(-VOCAB // (2 * PAIR))
VROWS = N_TBLK * 2 * PAIR


def _tp_body(a_ref, b_ref, o_ref):
    o_ref[:, 0:DIM] = jnp.transpose(a_ref[...])
    o_ref[:, DIM:2 * DIM] = jnp.transpose(b_ref[...])


_tc_transpose = pl.pallas_call(
    _tp_body,
    grid=(N_TBLK,),
    in_specs=[pl.BlockSpec((DIM, PAIR), lambda i: (0, jnp.minimum(2 * i, VOCAB // PAIR))),
              pl.BlockSpec((DIM, PAIR), lambda i: (0, jnp.minimum(2 * i + 1, VOCAB // PAIR)))],
    out_specs=pl.BlockSpec((PAIR, 2 * DIM), lambda i: (i, 0)),
    out_shape=jax.ShapeDtypeStruct((N_TBLK * PAIR, 2 * DIM), jnp.float32),
)


LBLK = B // 128


def _fmt_body(i_ref, o_ref):
    y = jnp.transpose(i_ref[...], (1, 2, 0))
    o_ref[...] = y.reshape(L, DIM // 8, 1, 8, 128)


_tc_format = pl.pallas_call(
    _fmt_body,
    grid=(LBLK,),
    compiler_params=pltpu.CompilerParams(vmem_limit_bytes=100 * 1024 * 1024),
    in_specs=[pl.BlockSpec((128, L // 2, 128), lambda i: (i, 0, 0))],
    out_specs=pl.BlockSpec((L, DIM // 8, 1, 8, 128), lambda i: (0, 0, i, 0, 0)),
    out_shape=jax.ShapeDtypeStruct((L, DIM // 8, LBLK, 8, 128), jnp.float32),
)


@functools.partial(
    pl.kernel,
    out_type=jax.ShapeDtypeStruct((TOTAL, DIM), jnp.float32),
    mesh=plsc.VectorSubcoreMesh(core_axis_name="c", subcore_axis_name="s"),
    compiler_params=pltpu.CompilerParams(use_tc_tiling_on_sc=False),
    scratch_types=(
        [pltpu.VMEM((PER_W,), jnp.int32)]
        + [pltpu.VMEM((CHUNK, DIM), jnp.float32) for _ in range(NBUF)]
        + [pltpu.SemaphoreType.DMA for _ in range(2 * NBUF)]
    ),
)
def _emb_gather(idx_hbm, table_hbm, out_hbm, idx_v, *rest):
    rows = rest[:NBUF]
    gsem = rest[NBUF:2 * NBUF]
    wsem = rest[2 * NBUF:3 * NBUF]

    wid = lax.axis_index("s") * NC + lax.axis_index("c")
    base = wid * PER_W

    pltpu.sync_copy(idx_hbm.at[pl.ds(base, PER_W)], idx_v)

    def g_copy(j, b):
        return pltpu.make_async_copy(
            table_hbm.at[idx_v.at[pl.ds(j * CHUNK, CHUNK)]], rows[b], gsem[b])

    def w_copy(j, b):
        return pltpu.make_async_copy(
            rows[b], out_hbm.at[pl.ds(base + j * CHUNK, CHUNK)], wsem[b])

    for b in range(NBUF):
        g_copy(b, b).start()

    zero16 = jnp.zeros((LANES,), jnp.float32)
    iota16 = lax.iota(jnp.int32, LANES)

    def _lane_min(r):
        for sh in (8, 4, 2, 1):
            perm = jnp.bitwise_xor(iota16, sh).reshape(LANES, 1)
            g = lax.gather(
                r, perm,
                lax.GatherDimensionNumbers(
                    offset_dims=(), collapsed_slice_dims=(0,),
                    start_index_map=(0,)),
                slice_sizes=(1,),
                mode=lax.GatherScatterMode.PROMISE_IN_BOUNDS)
            r = jnp.minimum(r, g)
        return r[0]

    def step(g, carry):
        for b in range(NBUF):
            j = g * NBUF + b
            g_copy(j, b).wait()
            groups = [idx_v[pl.ds(j * CHUNK + grp * LANES, LANES)]
                      for grp in range(CHUNK // LANES)]
            cmin = groups[0]
            for idxg in groups[1:]:
                cmin = jnp.minimum(cmin, idxg)

            @pl.when(_lane_min(cmin) == PAD_IDX)
            def _fix(j=j, b=b):
                def zero_group(grp, c):
                    idxg = idx_v[pl.ds(j * CHUNK + grp * LANES, LANES)]
                    for lane in range(LANES):
                        @pl.when(idxg[lane] == PAD_IDX)
                        def _(r=grp * LANES + lane):
                            for col in range(0, DIM, LANES):
                                rows[b][r, pl.ds(col, LANES)] = zero16
                    return c

                lax.fori_loop(0, CHUNK // LANES, zero_group, 0)
            w_copy(j, b).start()

            @pl.when(j < CHUNKS - NBUF)
            def _next(j=j, b=b):
                w_copy(j, b).wait()
                g_copy(j + NBUF, b).start()
        return carry

    lax.fori_loop(0, CHUNKS // NBUF, step, 0)
    for b in range(NBUF):
        w_copy(CHUNKS - NBUF + b, b).wait()


def kernel(source, table):
    tt = table.T
    tlin = _tc_transpose(tt, tt).reshape(VROWS, DIM)
    i = source.astype(jnp.int32)
    blk = i // (2 * PAIR)
    r = i % (2 * PAIR)
    v = blk * (2 * PAIR) + 2 * (r % PAIR) + r // PAIR
    idx = v.reshape(TOTAL)
    out = _emb_gather(idx, tlin)
    out5d = _tc_format(out.reshape(B, L // 2, 128))
    return out5d.transpose(2, 4, 0, 1, 3).reshape(B, L, DIM)
